# baseline (speedup 1.0000x reference)
import functools

import jax
import jax.numpy as jnp
from jax import lax
from jax.experimental import pallas as pl
from jax.experimental.pallas import tpu as pltpu
from jax.experimental.pallas import tpu_sc as plsc

NC = 2
NS = 16
_F32 = jnp.float32


def _deg_body(n, npad, blocks_per_tile, both2, ones_tab, zeros, out, idx2,
              ones_v, acc):
    cid = lax.axis_index("c")
    sid = lax.axis_index("s")
    rpt = (n // NS) & ~7
    rem_z = npad - NS * rpt
    rem_o = n - NS * rpt
    r0 = sid * rpt
    pltpu.sync_copy(ones_tab, ones_v)
    pltpu.sync_copy(zeros.at[pl.ds(r0, rpt)], acc.at[pl.ds(r0, rpt)])

    @pl.when(sid == 0)
    def _():
        pltpu.sync_copy(zeros.at[pl.ds(0, rem_z)],
                        acc.at[pl.ds(NS * rpt, rem_z)])

    plsc.subcore_barrier()

    @pl.loop(0, blocks_per_tile)
    def _(b):
        row = sid * (blocks_per_tile * 8) + b * 8
        pltpu.sync_copy(both2.at[cid, pl.ds(row, 8)], idx2)
        for j in range(8):
            pltpu.sync_copy(ones_v, acc.at[idx2.at[j]], add=True)

    plsc.subcore_barrier()
    pltpu.sync_copy(acc.at[pl.ds(r0, rpt)], out.at[cid, pl.ds(r0, rpt)])

    @pl.when(sid == 0)
    def _():
        pltpu.sync_copy(acc.at[pl.ds(NS * rpt, rem_o)],
                        out.at[cid, pl.ds(NS * rpt, rem_o)])


def _agg_body(n, npad, d, nblk, hpre, src2, dst2, zeros, out,
              idx_s, idx_d, rows, acc, sem_i, sem_g, sem_s0, sem_s1):
    cid = lax.axis_index("c")
    sid = lax.axis_index("s")
    rpt = (n // NS) & ~7
    rem_z = npad - NS * rpt
    rem_o = n - NS * rpt
    r0 = sid * rpt
    wid = cid * NS + sid
    row0 = wid * (nblk * 8)

    def idx_load(bb):
        slot = lax.rem(bb, 3)
        base = row0 + bb * 8
        pltpu.async_copy(src2.at[pl.ds(base, 8)], idx_s.at[slot], sem_i)
        pltpu.async_copy(dst2.at[pl.ds(base, 8)], idx_d.at[slot], sem_i)

    def idx_wait(bb):
        slot = lax.rem(bb, 3)
        base = row0 + bb * 8
        pltpu.make_async_copy(src2.at[pl.ds(base, 8)], idx_s.at[slot],
                              sem_i).wait()
        pltpu.make_async_copy(dst2.at[pl.ds(base, 8)], idx_d.at[slot],
                              sem_i).wait()

    sem_s = (sem_s0, sem_s1)

    def gather_start(slot, j, b):
        pltpu.async_copy(hpre.at[idx_s.at[slot, j]], rows.at[b], sem_g)

    def gather_wait(slot, j, b):
        pltpu.make_async_copy(hpre.at[idx_s.at[slot, j]], rows.at[b],
                              sem_g).wait()

    def scatter_start(slot, j, b):
        pltpu.async_copy(rows.at[b], acc.at[idx_d.at[slot, j]], sem_s[b],
                         add=True)

    def scatter_wait(slot, j, b):
        pltpu.make_async_copy(rows.at[b], acc.at[idx_d.at[slot, j]],
                              sem_s[b]).wait()

    idx_load(0)
    pltpu.sync_copy(zeros.at[pl.ds(r0, rpt)], acc.at[pl.ds(r0, rpt)])

    @pl.when(sid == 0)
    def _():
        pltpu.sync_copy(zeros.at[pl.ds(0, rem_z)],
                        acc.at[pl.ds(NS * rpt, rem_z)])

    plsc.subcore_barrier()

    @pl.loop(0, nblk)
    def _(bb):
        slot = lax.rem(bb, 3)
        prev = lax.rem(bb + 2, 3)

        @pl.when(bb + 1 < nblk)
        def _():
            idx_load(bb + 1)

        idx_wait(bb)
        for j in range(0, 8, 2):
            if j >= 2:
                scatter_wait(slot, j - 2, 0)
                scatter_wait(slot, j - 1, 1)
            else:
                @pl.when(bb > 0)
                def _():
                    scatter_wait(prev, 6, 0)
                    scatter_wait(prev, 7, 1)
            gather_start(slot, j, 0)
            gather_start(slot, j + 1, 1)
            gather_wait(slot, j, 0)
            gather_wait(slot, j + 1, 1)
            scatter_start(slot, j, 0)
            scatter_start(slot, j + 1, 1)

    last = (nblk - 1) % 3
    scatter_wait(last, 6, 0)
    scatter_wait(last, 7, 1)
    plsc.subcore_barrier()
    pltpu.sync_copy(acc.at[pl.ds(r0, rpt)], out.at[cid, pl.ds(r0, rpt)])

    @pl.when(sid == 0)
    def _():
        pltpu.sync_copy(acc.at[pl.ds(NS * rpt, rem_o)],
                        out.at[cid, pl.ds(NS * rpt, rem_o)])


def _prep_body(x_ref, dacc_ref, o_ref):
    n = x_ref.shape[0]
    nd = o_ref.shape[0] - n
    so = lax.rsqrt(jnp.maximum(dacc_ref[:, 0:1], 1.0))
    o_ref[pl.ds(0, n), :] = x_ref[...] * so
    o_ref[pl.ds(n, nd), :] = jnp.zeros((nd, x_ref.shape[1]), _F32)


def _layer1_body(p_ref, din_ref, dout_ref, w_ref, b_ref, o_ref):
    n = din_ref.shape[0]
    nd = o_ref.shape[0] - n
    m = p_ref[0] + p_ref[1]
    si = lax.rsqrt(jnp.maximum(din_ref[:, 0:1], 1.0))
    so = lax.rsqrt(jnp.maximum(dout_ref[:, 0:1], 1.0))
    t = jnp.dot(m * si, w_ref[...], preferred_element_type=_F32) + b_ref[...]
    h = jnp.where(t > 0, t, jnp.exp(jnp.minimum(t, 0.0)) - 1.0)
    o_ref[pl.ds(0, n), :] = h * so
    o_ref[pl.ds(n, nd), :] = jnp.zeros((nd, o_ref.shape[1]), _F32)


def _layer2_body(p_ref, din_ref, w_ref, b_ref, o_ref):
    m = p_ref[0] + p_ref[1]
    si = lax.rsqrt(jnp.maximum(din_ref[:, 0:1], 1.0))
    o_ref[...] = (jnp.dot(m * si, w_ref[...], preferred_element_type=_F32)
                  + b_ref[...])


def kernel(x, edge_index, W1, b1, W2, b2):
    n, d = x.shape
    e = edge_index.shape[1]
    ndum = 128
    npad = n + ndum
    unit = NC * NS * 1024
    e_pad = ((e + unit - 1) // unit) * unit
    pad = e_pad - e
    pad_i = jnp.arange(pad, dtype=jnp.int32)
    pad_dum = n + pad_i % ndum
    src2d = jnp.concatenate([edge_index[0], pad_dum]).reshape(e_pad // 128, 128)
    dst2 = jnp.concatenate([edge_index[1], pad_dum]).reshape(e_pad // 128, 128)
    src2 = jnp.concatenate([edge_index[0], pad_i % n]).reshape(e_pad // 128, 128)

    zeros = jnp.zeros((npad, d), _F32)
    ones_tab = jnp.ones((128, d), _F32)

    mesh = plsc.VectorSubcoreMesh(core_axis_name="c", subcore_axis_name="s")

    deg = pl.kernel(
        functools.partial(_deg_body, n, npad, e_pad // 128 // 8 // NS),
        out_type=jax.ShapeDtypeStruct((NC, n, d), _F32),
        mesh=mesh,
        scratch_types=[pltpu.VMEM((8, 128), jnp.int32),
                       pltpu.VMEM((128, d), _F32),
                       pltpu.VMEM_SHARED((npad, d), _F32)],
    )
    degs = deg(jnp.stack([src2d, dst2]), ones_tab, zeros)
    deg_s, deg_d = degs[0], degs[1]

    nblk = e_pad // 128 // 8 // (NC * NS)
    agg = pl.kernel(
        functools.partial(_agg_body, n, npad, d, nblk),
        out_type=jax.ShapeDtypeStruct((NC, n, d), _F32),
        mesh=mesh,
        scratch_types=[pltpu.VMEM((3, 8, 128), jnp.int32),
                       pltpu.VMEM((3, 8, 128), jnp.int32),
                       pltpu.VMEM((2, 128, d), _F32),
                       pltpu.VMEM_SHARED((npad, d), _F32),
                       pltpu.SemaphoreType.DMA,
                       pltpu.SemaphoreType.DMA,
                       pltpu.SemaphoreType.DMA,
                       pltpu.SemaphoreType.DMA],
    )

    hpre = pl.pallas_call(
        _prep_body,
        out_shape=jax.ShapeDtypeStruct((npad, d), _F32),
    )(x, deg_s)

    p1 = agg(hpre, src2, dst2, zeros)

    hpre2 = pl.pallas_call(
        _layer1_body,
        out_shape=jax.ShapeDtypeStruct((npad, d), _F32),
    )(p1, deg_d, deg_s, W1, b1)

    p2 = agg(hpre2, src2, dst2, zeros)

    out = pl.pallas_call(
        _layer2_body,
        out_shape=jax.ShapeDtypeStruct((n, d), _F32),
    )(p2, deg_d, W2, b2)
    return out


# speedup vs baseline: 1.1155x; 1.1155x over previous
import functools

import jax
import jax.numpy as jnp
from jax import lax
from jax.experimental import pallas as pl
from jax.experimental.pallas import tpu as pltpu
from jax.experimental.pallas import tpu_sc as plsc

NC = 2
NS = 16
_F32 = jnp.float32


def _deg_body(n, npad, blocks_per_tile, both2, ones_tab, zeros, out, idx2,
              ones_v, acc, sem_i, sem_s):
    cid = lax.axis_index("c")
    sid = lax.axis_index("s")
    rpt = (n // NS) & ~7
    rem_z = npad - NS * rpt
    rem_o = n - NS * rpt
    r0 = sid * rpt
    pltpu.sync_copy(ones_tab, ones_v)
    pltpu.sync_copy(zeros.at[pl.ds(r0, rpt)], acc.at[pl.ds(r0, rpt)])

    @pl.when(sid == 0)
    def _():
        pltpu.sync_copy(zeros.at[pl.ds(0, rem_z)],
                        acc.at[pl.ds(NS * rpt, rem_z)])

    plsc.subcore_barrier()

    row0 = sid * (blocks_per_tile * 8)

    def idx_load(bb):
        slot = lax.rem(bb, 3)
        pltpu.async_copy(both2.at[cid, pl.ds(row0 + bb * 8, 8)],
                         idx2.at[slot], sem_i)

    def idx_wait(bb):
        slot = lax.rem(bb, 3)
        pltpu.make_async_copy(both2.at[cid, pl.ds(row0 + bb * 8, 8)],
                              idx2.at[slot], sem_i).wait()

    def sc_start(slot, j):
        pltpu.async_copy(ones_v, acc.at[idx2.at[slot, j]], sem_s, add=True)

    def sc_wait(slot, j):
        pltpu.make_async_copy(ones_v, acc.at[idx2.at[slot, j]], sem_s).wait()

    idx_load(0)

    @pl.loop(0, blocks_per_tile)
    def _(bb):
        slot = lax.rem(bb, 3)
        prev = lax.rem(bb + 2, 3)

        @pl.when(bb + 1 < blocks_per_tile)
        def _():
            idx_load(bb + 1)

        @pl.when(bb > 0)
        def _():
            for j in range(8):
                sc_wait(prev, j)

        idx_wait(bb)
        for j in range(8):
            sc_start(slot, j)

    last = (blocks_per_tile - 1) % 3
    for j in range(8):
        sc_wait(last, j)

    plsc.subcore_barrier()
    pltpu.sync_copy(acc.at[pl.ds(r0, rpt)], out.at[cid, pl.ds(r0, rpt)])

    @pl.when(sid == 0)
    def _():
        pltpu.sync_copy(acc.at[pl.ds(NS * rpt, rem_o)],
                        out.at[cid, pl.ds(NS * rpt, rem_o)])


def _agg_body(n, npad, d, nblk, hpre, src2, dst2, zeros, out,
              idx_s, idx_d, rows, acc, sem_i, sem_g, sem_s0, sem_s1):
    cid = lax.axis_index("c")
    sid = lax.axis_index("s")
    rpt = (n // NS) & ~7
    rem_z = npad - NS * rpt
    rem_o = n - NS * rpt
    r0 = sid * rpt
    wid = cid * NS + sid
    row0 = wid * (nblk * 8)

    def idx_load(bb):
        slot = lax.rem(bb, 3)
        base = row0 + bb * 8
        pltpu.async_copy(src2.at[pl.ds(base, 8)], idx_s.at[slot], sem_i)
        pltpu.async_copy(dst2.at[pl.ds(base, 8)], idx_d.at[slot], sem_i)

    def idx_wait(bb):
        slot = lax.rem(bb, 3)
        base = row0 + bb * 8
        pltpu.make_async_copy(src2.at[pl.ds(base, 8)], idx_s.at[slot],
                              sem_i).wait()
        pltpu.make_async_copy(dst2.at[pl.ds(base, 8)], idx_d.at[slot],
                              sem_i).wait()

    sem_s = (sem_s0, sem_s1)

    def gather_start(slot, j, b):
        pltpu.async_copy(hpre.at[idx_s.at[slot, j]], rows.at[b], sem_g)

    def gather_wait(slot, j, b):
        pltpu.make_async_copy(hpre.at[idx_s.at[slot, j]], rows.at[b],
                              sem_g).wait()

    def scatter_start(slot, j, b):
        pltpu.async_copy(rows.at[b], acc.at[idx_d.at[slot, j]], sem_s[b],
                         add=True)

    def scatter_wait(slot, j, b):
        pltpu.make_async_copy(rows.at[b], acc.at[idx_d.at[slot, j]],
                              sem_s[b]).wait()

    idx_load(0)
    pltpu.sync_copy(zeros.at[pl.ds(r0, rpt)], acc.at[pl.ds(r0, rpt)])

    @pl.when(sid == 0)
    def _():
        pltpu.sync_copy(zeros.at[pl.ds(0, rem_z)],
                        acc.at[pl.ds(NS * rpt, rem_z)])

    plsc.subcore_barrier()

    @pl.loop(0, nblk)
    def _(bb):
        slot = lax.rem(bb, 3)
        prev = lax.rem(bb + 2, 3)

        @pl.when(bb + 1 < nblk)
        def _():
            idx_load(bb + 1)

        idx_wait(bb)
        for j in range(8):
            b = j & 1
            if j >= 2:
                scatter_wait(slot, j - 2, b)
            else:
                @pl.when(bb > 0)
                def _():
                    scatter_wait(prev, j + 6, b)
            gather_start(slot, j, b)
            gather_wait(slot, j, b)
            scatter_start(slot, j, b)

    last = (nblk - 1) % 3
    scatter_wait(last, 6, 0)
    scatter_wait(last, 7, 1)
    plsc.subcore_barrier()
    pltpu.sync_copy(acc.at[pl.ds(r0, rpt)], out.at[cid, pl.ds(r0, rpt)])

    @pl.when(sid == 0)
    def _():
        pltpu.sync_copy(acc.at[pl.ds(NS * rpt, rem_o)],
                        out.at[cid, pl.ds(NS * rpt, rem_o)])


def _prep_body(x_ref, dacc_ref, o_ref):
    n = x_ref.shape[0]
    nd = o_ref.shape[0] - n
    so = lax.rsqrt(jnp.maximum(dacc_ref[:, 0:1], 1.0))
    o_ref[pl.ds(0, n), :] = x_ref[...] * so
    o_ref[pl.ds(n, nd), :] = jnp.zeros((nd, x_ref.shape[1]), _F32)


def _layer1_body(p_ref, din_ref, dout_ref, w_ref, b_ref, o_ref):
    n = din_ref.shape[0]
    nd = o_ref.shape[0] - n
    m = p_ref[0] + p_ref[1]
    si = lax.rsqrt(jnp.maximum(din_ref[:, 0:1], 1.0))
    so = lax.rsqrt(jnp.maximum(dout_ref[:, 0:1], 1.0))
    t = jnp.dot(m * si, w_ref[...], preferred_element_type=_F32) + b_ref[...]
    h = jnp.where(t > 0, t, jnp.exp(jnp.minimum(t, 0.0)) - 1.0)
    o_ref[pl.ds(0, n), :] = h * so
    o_ref[pl.ds(n, nd), :] = jnp.zeros((nd, o_ref.shape[1]), _F32)


def _layer2_body(p_ref, din_ref, w_ref, b_ref, o_ref):
    m = p_ref[0] + p_ref[1]
    si = lax.rsqrt(jnp.maximum(din_ref[:, 0:1], 1.0))
    o_ref[...] = (jnp.dot(m * si, w_ref[...], preferred_element_type=_F32)
                  + b_ref[...])


def kernel(x, edge_index, W1, b1, W2, b2):
    n, d = x.shape
    e = edge_index.shape[1]
    ndum = 128
    npad = n + ndum
    unit = NC * NS * 1024
    e_pad = ((e + unit - 1) // unit) * unit
    pad = e_pad - e
    pad_i = jnp.arange(pad, dtype=jnp.int32)
    pad_dum = n + pad_i % ndum
    src2d = jnp.concatenate([edge_index[0], pad_dum]).reshape(e_pad // 128, 128)
    dst2 = jnp.concatenate([edge_index[1], pad_dum]).reshape(e_pad // 128, 128)
    src2 = jnp.concatenate([edge_index[0], pad_i % n]).reshape(e_pad // 128, 128)

    zeros = jnp.zeros((npad, d), _F32)
    ones_tab = jnp.ones((128, d), _F32)

    mesh = plsc.VectorSubcoreMesh(core_axis_name="c", subcore_axis_name="s")

    deg = pl.kernel(
        functools.partial(_deg_body, n, npad, e_pad // 128 // 8 // NS),
        out_type=jax.ShapeDtypeStruct((NC, n, d), _F32),
        mesh=mesh,
        scratch_types=[pltpu.VMEM((3, 8, 128), jnp.int32),
                       pltpu.VMEM((128, d), _F32),
                       pltpu.VMEM_SHARED((npad, d), _F32),
                       pltpu.SemaphoreType.DMA,
                       pltpu.SemaphoreType.DMA],
    )
    degs = deg(jnp.stack([src2d, dst2]), ones_tab, zeros)
    deg_s, deg_d = degs[0], degs[1]

    nblk = e_pad // 128 // 8 // (NC * NS)
    agg = pl.kernel(
        functools.partial(_agg_body, n, npad, d, nblk),
        out_type=jax.ShapeDtypeStruct((NC, n, d), _F32),
        mesh=mesh,
        scratch_types=[pltpu.VMEM((3, 8, 128), jnp.int32),
                       pltpu.VMEM((3, 8, 128), jnp.int32),
                       pltpu.VMEM((2, 128, d), _F32),
                       pltpu.VMEM_SHARED((npad, d), _F32),
                       pltpu.SemaphoreType.DMA,
                       pltpu.SemaphoreType.DMA,
                       pltpu.SemaphoreType.DMA,
                       pltpu.SemaphoreType.DMA],
    )

    hpre = pl.pallas_call(
        _prep_body,
        out_shape=jax.ShapeDtypeStruct((npad, d), _F32),
    )(x, deg_s)

    p1 = agg(hpre, src2, dst2, zeros)

    hpre2 = pl.pallas_call(
        _layer1_body,
        out_shape=jax.ShapeDtypeStruct((npad, d), _F32),
    )(p1, deg_d, deg_s, W1, b1)

    p2 = agg(hpre2, src2, dst2, zeros)

    out = pl.pallas_call(
        _layer2_body,
        out_shape=jax.ShapeDtypeStruct((n, d), _F32),
    )(p2, deg_d, W2, b2)
    return out
